# baseline (speedup 1.0000x reference)
import functools

import jax
import jax.numpy as jnp
from jax import lax
from jax.experimental import pallas as pl
from jax.experimental.pallas import tpu as pltpu
from jax.experimental.pallas import tpu_sc as plsc

N = 2048
D = 512
K = 5
BLK = 1024
NSEL = 16
MARGIN = 1.0
ALPHA = 0.3

NW = 32
ROWS_PER_W = N // NW
RB = 8
L = 16


def _sq_kernel(a_ref, p_ref, ng_ref, er_ref, e_ref, sq_ref, tri_ref,
               eaug_ref):
    i = pl.program_id(0)
    ones_d = jnp.ones((D, 1), jnp.float32)

    @pl.when(i == 0)
    def _init():
        tri_ref[...] = jnp.zeros((1, 1), jnp.float32)
        e0 = e_ref[...]
        cn = jax.lax.dot_general(e0 * e0, ones_d, (((1,), (0,)), ((), ())),
                                 preferred_element_type=jnp.float32)
        eaug_ref[:, :D] = e0 * (-2.0)
        eaug_ref[:, D:D + 1] = cn
        eaug_ref[:, D + 1:D + 2] = jnp.ones((N, 1), jnp.float32)

    a = a_ref[...]
    dp = a - p_ref[...] + 1e-6
    dn = a - ng_ref[...] + 1e-6
    pos_d = jnp.sqrt(jax.lax.dot_general(dp * dp, ones_d,
                                         (((1,), (0,)), ((), ())),
                                         preferred_element_type=jnp.float32))
    neg_d = jnp.sqrt(jax.lax.dot_general(dn * dn, ones_d,
                                         (((1,), (0,)), ((), ())),
                                         preferred_element_type=jnp.float32))
    tri_ref[...] += jnp.sum(jnp.maximum(pos_d - neg_d + MARGIN, 0.0),
                            keepdims=True).reshape(1, 1)

    rows = er_ref[...]
    rn = jax.lax.dot_general(rows * rows, ones_d, (((1,), (0,)), ((), ())),
                             preferred_element_type=jnp.float32)
    rows_aug = jnp.concatenate(
        [rows, jnp.ones((BLK, 1), jnp.float32), rn], axis=1)
    sq_ref[...] = jax.lax.dot_general(rows_aug, eaug_ref[...],
                                      (((1,), (1,)), ((), ())),
                                      preferred_element_type=jnp.float32)


def _tc_stage(anchor, positive, negative, embeddings):
    return pl.pallas_call(
        _sq_kernel,
        grid=(N // BLK,),
        in_specs=[
            pl.BlockSpec((BLK, D), lambda i: (i, 0)),
            pl.BlockSpec((BLK, D), lambda i: (i, 0)),
            pl.BlockSpec((BLK, D), lambda i: (i, 0)),
            pl.BlockSpec((BLK, D), lambda i: (i, 0)),
            pl.BlockSpec((N, D), lambda i: (0, 0)),
        ],
        out_specs=[
            pl.BlockSpec((BLK, N), lambda i: (i, 0)),
            pl.BlockSpec((1, 1), lambda i: (0, 0)),
        ],
        out_shape=[
            jax.ShapeDtypeStruct((N, N), jnp.float32),
            jax.ShapeDtypeStruct((1, 1), jnp.float32),
        ],
        scratch_shapes=[pltpu.VMEM((N, D + 2), jnp.float32)],
    )(anchor, positive, negative, embeddings, embeddings)


def _splat(x):
    return jnp.full((L,), x, jnp.float32)


def _gather16(x, idx):
    dnums = lax.GatherDimensionNumbers(offset_dims=(),
                                       collapsed_slice_dims=(0,),
                                       start_index_map=(0,))
    return lax.gather(x, idx[:, None], dnums, (1,),
                      mode=lax.GatherScatterMode.PROMISE_IN_BOUNDS)


def _rot(x, k):
    idx = (lax.iota(jnp.int32, L) + k) & (L - 1)
    return _gather16(x, idx)


def _rotmin(x):
    for k in (8, 4, 2, 1):
        x = jnp.minimum(x, _rot(x, k))
    return x


def _rotsum(x):
    for k in (8, 4, 2, 1):
        x = x + _rot(x, k)
    return x


def _prefix(x):
    lane = lax.iota(jnp.int32, L)
    for k in (1, 2, 4, 8):
        sh = _gather16(x, jnp.maximum(lane - k, 0))
        x = x + jnp.where(lane >= k, sh, 0.0)
    return x


def _nsqrt(x):
    i = lax.bitcast_convert_type(x, jnp.int32)
    y = lax.bitcast_convert_type(
        jnp.int32(0x1FBD1DF5) + lax.shift_right_logical(i, 1), jnp.float32)
    for _ in range(3):
        y = 0.5 * (y + x / y)
    return y


def _sc_body(sq_hbm, out_hbm, rowbuf, accbuf):
    c = lax.axis_index("c")
    s = lax.axis_index("s")
    wid = s * 2 + c
    base = wid * ROWS_PER_W

    inf = jnp.full((L,), jnp.inf, jnp.float32)

    def row_body(r, acc):
        row_g = base + r
        pltpu.sync_copy(sq_hbm.at[pl.ds(row_g, 1)], rowbuf)

        def chunk(ci, carry):
            st = list(carry)
            for u in range(4):
                x = rowbuf[0, pl.ds((4 * ci + u) * L, L)]
                for lvl in range(K + 1):
                    lo = jnp.minimum(st[lvl], x)
                    x = jnp.maximum(st[lvl], x)
                    st[lvl] = lo
            return tuple(st)

        st = lax.fori_loop(0, (N // L) // 4, chunk,
                           tuple(inf for _ in range(K + 1)))
        work = list(st)

        rank_before = _splat(0.0)
        t_sq = _splat(0.0)
        count_less = _splat(0.0)
        nb6_sum = _splat(0.0)
        min_dm = _splat(0.0)
        for t in range(K + 1):
            mv = work[0]
            for w_ in work[1:]:
                mv = jnp.minimum(mv, w_)
            m = _rotmin(mv)
            cv = jnp.zeros((L,), jnp.float32)
            for w_ in work:
                cv = cv + jnp.where(w_ == m, 1.0, 0.0)
            cnt = _rotsum(cv)
            k = jnp.clip(_splat(float(K + 1)) - rank_before, 0.0, cnt)
            dm_t = _nsqrt(jnp.maximum(m, 1e-12))
            nb6_sum = nb6_sum + jnp.where(k > 0.0, k * dm_t, 0.0)
            if t == 0:
                min_dm = dm_t
            c_lo = jnp.where(rank_before < float(K + 1), 1.0, 0.0)
            c_hi = jnp.where(rank_before + cnt >= float(K + 1), 1.0, 0.0)
            c_f = c_lo * c_hi
            t_sq = t_sq + c_f * (m - t_sq)
            count_less = count_less + c_f * (rank_before - count_less)
            rank_before = rank_before + cnt
            if t < K:
                work = [jnp.where(w_ == m, jnp.inf, w_) for w_ in work]
        nb_sum = nb6_sum - min_dm

        sq16 = rowbuf[0, pl.ds(0, L)]
        dm16 = _nsqrt(jnp.maximum(sq16, 1e-12))
        col16 = lax.iota(jnp.int32, L)
        is_tie = jnp.where(sq16 == t_sq, 1.0, 0.0)
        tie_prefix = _prefix(is_tie) - is_tie
        lt_f = jnp.where(sq16 < t_sq, 1.0, 0.0)
        tie_ok = jnp.where(tie_prefix + count_less < float(K + 1), 1.0, 0.0)
        in_top6_f = jnp.minimum(lt_f + is_tie * tie_ok, 1.0)
        row_gv = jnp.full((L,), row_g, jnp.int32)
        is_self = jnp.where(col16 == row_gv, 1.0, 0.0)
        valid_f = (1.0 - is_self) * (1.0 - in_top6_f)
        rank = _prefix(valid_f)
        take_f = valid_f * jnp.where(rank <= float(K), 1.0, 0.0)
        nn_sum = _rotsum(take_f * dm16)

        loc = jnp.maximum(nb_sum / K - nn_sum / K + 0.1, 0.0)
        return acc + loc

    acc = lax.fori_loop(0, ROWS_PER_W, row_body, jnp.zeros((L,), jnp.float32))
    accbuf[...] = acc * (1.0 / L)
    pltpu.sync_copy(accbuf, out_hbm.at[wid])


def _sc_stage(sqm):
    mesh = plsc.VectorSubcoreMesh(core_axis_name="c", subcore_axis_name="s")
    f = functools.partial(
        pl.kernel,
        out_type=jax.ShapeDtypeStruct((NW, L), jnp.float32),
        mesh=mesh,
        scratch_types=[
            pltpu.VMEM((1, N), jnp.float32),
            pltpu.VMEM((L,), jnp.float32),
        ],
    )(_sc_body)
    return f(sqm)


def kernel(anchor, positive, negative, embeddings):
    sqm, tri_sum = _tc_stage(anchor, positive, negative, embeddings)
    part = _sc_stage(sqm)
    return tri_sum[0, 0] / N + ALPHA * (jnp.sum(part) / N)


# speedup vs baseline: 3.9181x; 3.9181x over previous
import jax
import jax.numpy as jnp
from jax.experimental import pallas as pl
from jax.experimental.pallas import tpu as pltpu

N = 2048
D = 512
K = 5
BLK = 1024
CHK = 256
SUB = 8
NSEL = 16
MARGIN = 1.0
ALPHA = 0.3


def _ce(a, b):
    return jnp.minimum(a, b), jnp.maximum(a, b)


def _bitonic6(L):
    L = list(L)
    for i in range(3):
        L[i], L[i + 3] = _ce(L[i], L[i + 3])
    for base in (0, 3):
        L[base], L[base + 2] = _ce(L[base], L[base + 2])
        L[base], L[base + 1] = _ce(L[base], L[base + 1])
        L[base + 1], L[base + 2] = _ce(L[base + 1], L[base + 2])
    return L


def _merge66(A, B):
    return _bitonic6([jnp.minimum(A[i], B[5 - i]) for i in range(6)])


def _merge44(A, B):
    return _bitonic6([A[0], A[1], jnp.minimum(A[2], B[3]),
                      jnp.minimum(A[3], B[2]), B[1], B[0]])


def _merge22(A, B):
    lo0, hi0 = _ce(A[0], B[0])
    lo1, hi1 = _ce(A[1], B[1])
    mid_lo, mid_hi = _ce(hi0, lo1)
    return [lo0, mid_lo, mid_hi, hi1]


def _top6_sorted_T(sq_t):
    groups = [sq_t[g * SUB:(g + 1) * SUB, :] for g in range(N // SUB)]
    lists = []
    for p in range(len(groups) // 2):
        lo, hi = _ce(groups[2 * p], groups[2 * p + 1])
        lists.append([lo, hi])
    lists = [_merge22(lists[2 * p], lists[2 * p + 1])
             for p in range(len(lists) // 2)]
    lists = [_merge44(lists[2 * p], lists[2 * p + 1])
             for p in range(len(lists) // 2)]
    while len(lists) > 1:
        lists = [_merge66(lists[2 * p], lists[2 * p + 1])
                 for p in range(len(lists) // 2)]
    s = lists[0]
    w = SUB // 2
    while w >= 1:
        s = _merge66([v[:w, :] for v in s], [v[w:2 * w, :] for v in s])
        w //= 2
    return s


def _loss_kernel(a_ref, p_ref, ng_ref, er_ref, e_ref, tri_ref, loc_ref,
                 eaug_ref):
    i = pl.program_id(0)
    ones_d = jnp.ones((D, 1), jnp.float32)

    @pl.when(i == 0)
    def _init():
        tri_ref[...] = jnp.zeros((1, 1), jnp.float32)
        loc_ref[...] = jnp.zeros((1, 1), jnp.float32)
        e0 = e_ref[...]
        cn = jax.lax.dot_general(e0 * e0, ones_d, (((1,), (0,)), ((), ())),
                                 preferred_element_type=jnp.float32)
        eaug_ref[:, :D] = e0 * (-2.0)
        eaug_ref[:, D:D + 1] = cn
        eaug_ref[:, D + 1:D + 2] = jnp.ones((N, 1), jnp.float32)

    a = a_ref[...]
    dp = a - p_ref[...] + 1e-6
    dn = a - ng_ref[...] + 1e-6
    pos_d = jnp.sqrt(jax.lax.dot_general(dp * dp, ones_d,
                                         (((1,), (0,)), ((), ())),
                                         preferred_element_type=jnp.float32))
    neg_d = jnp.sqrt(jax.lax.dot_general(dn * dn, ones_d,
                                         (((1,), (0,)), ((), ())),
                                         preferred_element_type=jnp.float32))
    tri_ref[...] += jnp.sum(jnp.maximum(pos_d - neg_d + MARGIN, 0.0),
                            keepdims=True).reshape(1, 1)

    rows = er_ref[...]
    rn = jax.lax.dot_general(rows * rows, ones_d, (((1,), (0,)), ((), ())),
                             preferred_element_type=jnp.float32)
    rows_aug = jnp.concatenate(
        [rows, jnp.ones((BLK, 1), jnp.float32), rn], axis=1)
    eaug = eaug_ref[...]
    loc_acc = jnp.zeros((1, 1), jnp.float32)
    for cc in range(BLK // CHK):
        ra = rows_aug[cc * CHK:(cc + 1) * CHK, :]
        sq_t = jax.lax.dot_general(eaug, ra, (((1,), (1,)), ((), ())),
                                   preferred_element_type=jnp.float32)
        loc_acc += _local_chunk(sq_t, i * BLK + cc * CHK)
    loc_ref[...] += loc_acc


def _local_chunk(sq_t, col0):
    s6 = _top6_sorted_T(sq_t)
    dm6 = [jnp.sqrt(jnp.maximum(v, 1e-12)) for v in s6]
    nb_sum = dm6[1] + dm6[2] + dm6[3] + dm6[4] + dm6[5]
    t_sq = s6[5]
    count_less = ((s6[0] < t_sq).astype(jnp.float32)
                  + (s6[1] < t_sq).astype(jnp.float32)
                  + (s6[2] < t_sq).astype(jnp.float32)
                  + (s6[3] < t_sq).astype(jnp.float32)
                  + (s6[4] < t_sq).astype(jnp.float32))

    sq16 = sq_t[:NSEL, :]
    dm16 = jnp.sqrt(jnp.maximum(sq16, 1e-12))
    col16 = jax.lax.broadcasted_iota(jnp.int32, (NSEL, CHK), 0)
    row_idx = col0 + jax.lax.broadcasted_iota(jnp.int32, (1, CHK), 1)
    is_tie = (sq16 == t_sq).astype(jnp.float32)
    strict_lo = (jax.lax.broadcasted_iota(jnp.int32, (NSEL, NSEL), 1)
                 < jax.lax.broadcasted_iota(jnp.int32, (NSEL, NSEL), 0))
    tie_prefix = jax.lax.dot_general(strict_lo.astype(jnp.float32), is_tie,
                                     (((1,), (0,)), ((), ())),
                                     preferred_element_type=jnp.float32)
    in_top6 = (sq16 < t_sq) | ((sq16 == t_sq)
                               & (tie_prefix + count_less < K + 1))
    valid = ~((col16 == row_idx) | in_top6)
    incl_lo = (jax.lax.broadcasted_iota(jnp.int32, (NSEL, NSEL), 1)
               <= jax.lax.broadcasted_iota(jnp.int32, (NSEL, NSEL), 0))
    rank = jax.lax.dot_general(incl_lo.astype(jnp.float32),
                               valid.astype(jnp.float32),
                               (((1,), (0,)), ((), ())),
                               preferred_element_type=jnp.float32)
    take = valid & (rank <= float(K))
    nn_sum = jax.lax.dot_general(jnp.ones((1, NSEL), jnp.float32),
                                 jnp.where(take, dm16, 0.0),
                                 (((1,), (0,)), ((), ())),
                                 preferred_element_type=jnp.float32)

    loc = jnp.maximum(nb_sum / K - nn_sum / K + 0.1, 0.0)
    return jnp.sum(loc, keepdims=True).reshape(1, 1)


def kernel(anchor, positive, negative, embeddings):
    grid = (N // BLK,)
    tri_sum, loc_sum = pl.pallas_call(
        _loss_kernel,
        grid=grid,
        in_specs=[
            pl.BlockSpec((BLK, D), lambda i: (i, 0)),
            pl.BlockSpec((BLK, D), lambda i: (i, 0)),
            pl.BlockSpec((BLK, D), lambda i: (i, 0)),
            pl.BlockSpec((BLK, D), lambda i: (i, 0)),
            pl.BlockSpec((N, D), lambda i: (0, 0)),
        ],
        out_specs=[
            pl.BlockSpec((1, 1), lambda i: (0, 0)),
            pl.BlockSpec((1, 1), lambda i: (0, 0)),
        ],
        out_shape=[
            jax.ShapeDtypeStruct((1, 1), jnp.float32),
            jax.ShapeDtypeStruct((1, 1), jnp.float32),
        ],
        scratch_shapes=[pltpu.VMEM((N, D + 2), jnp.float32)],
    )(anchor, positive, negative, embeddings, embeddings)
    return tri_sum[0, 0] / N + ALPHA * (loc_sum[0, 0] / N)


# speedup vs baseline: 5.0388x; 1.2860x over previous
import jax
import jax.numpy as jnp
from jax.experimental import pallas as pl
from jax.experimental.pallas import tpu as pltpu

N = 2048
D = 512
K = 5
BLK = 1024
CHK = 512
SUB = 8
NSEL = 16
MARGIN = 1.0
ALPHA = 0.3


def _ce(a, b):
    return jnp.minimum(a, b), jnp.maximum(a, b)


def _bitonic6(L):
    L = list(L)
    for i in range(3):
        L[i], L[i + 3] = _ce(L[i], L[i + 3])
    for base in (0, 3):
        L[base], L[base + 2] = _ce(L[base], L[base + 2])
        L[base], L[base + 1] = _ce(L[base], L[base + 1])
        L[base + 1], L[base + 2] = _ce(L[base + 1], L[base + 2])
    return L


def _merge66(A, B):
    return _bitonic6([jnp.minimum(A[i], B[5 - i]) for i in range(6)])


def _merge44(A, B):
    return _bitonic6([A[0], A[1], jnp.minimum(A[2], B[3]),
                      jnp.minimum(A[3], B[2]), B[1], B[0]])


def _merge22(A, B):
    lo0, hi0 = _ce(A[0], B[0])
    lo1, hi1 = _ce(A[1], B[1])
    mid_lo, mid_hi = _ce(hi0, lo1)
    return [lo0, mid_lo, mid_hi, hi1]


def _top6_sorted_T(sq_t):
    groups = [sq_t[g * SUB:(g + 1) * SUB, :] for g in range(N // SUB)]
    lists = []
    for p in range(len(groups) // 2):
        lo, hi = _ce(groups[2 * p], groups[2 * p + 1])
        lists.append([lo, hi])
    lists = [_merge22(lists[2 * p], lists[2 * p + 1])
             for p in range(len(lists) // 2)]
    lists = [_merge44(lists[2 * p], lists[2 * p + 1])
             for p in range(len(lists) // 2)]
    while len(lists) > 1:
        lists = [_merge66(lists[2 * p], lists[2 * p + 1])
                 for p in range(len(lists) // 2)]
    s = lists[0]
    w = SUB // 2
    while w >= 1:
        s = _merge66([v[:w, :] for v in s], [v[w:2 * w, :] for v in s])
        w //= 2
    return s


def _loss_kernel(a_ref, p_ref, ng_ref, er_ref, e_ref, tri_ref, loc_ref,
                 eaug_ref):
    i = pl.program_id(0)
    ones_d = jnp.ones((D, 1), jnp.float32)

    @pl.when(i == 0)
    def _init():
        tri_ref[...] = jnp.zeros((1, 1), jnp.float32)
        loc_ref[...] = jnp.zeros((1, 1), jnp.float32)
        e0 = e_ref[...]
        cn = jax.lax.dot_general(e0 * e0, ones_d, (((1,), (0,)), ((), ())),
                                 preferred_element_type=jnp.float32)
        eaug_ref[:, :D] = e0 * (-2.0)
        eaug_ref[:, D:D + 1] = cn
        eaug_ref[:, D + 1:D + 2] = jnp.ones((N, 1), jnp.float32)

    a = a_ref[...]
    dp = a - p_ref[...] + 1e-6
    dn = a - ng_ref[...] + 1e-6
    pos_d = jnp.sqrt(jax.lax.dot_general(dp * dp, ones_d,
                                         (((1,), (0,)), ((), ())),
                                         preferred_element_type=jnp.float32))
    neg_d = jnp.sqrt(jax.lax.dot_general(dn * dn, ones_d,
                                         (((1,), (0,)), ((), ())),
                                         preferred_element_type=jnp.float32))
    tri_ref[...] += jnp.sum(jnp.maximum(pos_d - neg_d + MARGIN, 0.0),
                            keepdims=True).reshape(1, 1)

    rows = er_ref[...]
    rn = jax.lax.dot_general(rows * rows, ones_d, (((1,), (0,)), ((), ())),
                             preferred_element_type=jnp.float32)
    rows_aug = jnp.concatenate(
        [rows, jnp.ones((BLK, 1), jnp.float32), rn], axis=1)
    eaug = eaug_ref[...]
    loc_acc = jnp.zeros((1, 1), jnp.float32)
    for cc in range(BLK // CHK):
        ra = rows_aug[cc * CHK:(cc + 1) * CHK, :]
        sq_t = jax.lax.dot_general(eaug, ra, (((1,), (1,)), ((), ())),
                                   preferred_element_type=jnp.float32)
        loc_acc += _local_chunk(sq_t, i * BLK + cc * CHK)
    loc_ref[...] += loc_acc


def _local_chunk(sq_t, col0):
    s6 = _top6_sorted_T(sq_t)
    dm6 = [jnp.sqrt(jnp.maximum(v, 1e-12)) for v in s6]
    nb_sum = dm6[1] + dm6[2] + dm6[3] + dm6[4] + dm6[5]
    t_sq = s6[5]
    count_less = ((s6[0] < t_sq).astype(jnp.float32)
                  + (s6[1] < t_sq).astype(jnp.float32)
                  + (s6[2] < t_sq).astype(jnp.float32)
                  + (s6[3] < t_sq).astype(jnp.float32)
                  + (s6[4] < t_sq).astype(jnp.float32))

    sq16 = sq_t[:NSEL, :]
    dm16 = jnp.sqrt(jnp.maximum(sq16, 1e-12))
    col16 = jax.lax.broadcasted_iota(jnp.int32, (NSEL, CHK), 0)
    row_idx = col0 + jax.lax.broadcasted_iota(jnp.int32, (1, CHK), 1)
    is_tie = (sq16 == t_sq).astype(jnp.float32)
    strict_lo = (jax.lax.broadcasted_iota(jnp.int32, (NSEL, NSEL), 1)
                 < jax.lax.broadcasted_iota(jnp.int32, (NSEL, NSEL), 0))
    tie_prefix = jax.lax.dot_general(strict_lo.astype(jnp.float32), is_tie,
                                     (((1,), (0,)), ((), ())),
                                     preferred_element_type=jnp.float32)
    in_top6 = (sq16 < t_sq) | ((sq16 == t_sq)
                               & (tie_prefix + count_less < K + 1))
    valid = ~((col16 == row_idx) | in_top6)
    incl_lo = (jax.lax.broadcasted_iota(jnp.int32, (NSEL, NSEL), 1)
               <= jax.lax.broadcasted_iota(jnp.int32, (NSEL, NSEL), 0))
    rank = jax.lax.dot_general(incl_lo.astype(jnp.float32),
                               valid.astype(jnp.float32),
                               (((1,), (0,)), ((), ())),
                               preferred_element_type=jnp.float32)
    take = valid & (rank <= float(K))
    nn_sum = jax.lax.dot_general(jnp.ones((1, NSEL), jnp.float32),
                                 jnp.where(take, dm16, 0.0),
                                 (((1,), (0,)), ((), ())),
                                 preferred_element_type=jnp.float32)

    loc = jnp.maximum(nb_sum / K - nn_sum / K + 0.1, 0.0)
    return jnp.sum(loc, keepdims=True).reshape(1, 1)


def kernel(anchor, positive, negative, embeddings):
    grid = (N // BLK,)
    tri_sum, loc_sum = pl.pallas_call(
        _loss_kernel,
        grid=grid,
        in_specs=[
            pl.BlockSpec((BLK, D), lambda i: (i, 0)),
            pl.BlockSpec((BLK, D), lambda i: (i, 0)),
            pl.BlockSpec((BLK, D), lambda i: (i, 0)),
            pl.BlockSpec((BLK, D), lambda i: (i, 0)),
            pl.BlockSpec((N, D), lambda i: (0, 0)),
        ],
        out_specs=[
            pl.BlockSpec((1, 1), lambda i: (0, 0)),
            pl.BlockSpec((1, 1), lambda i: (0, 0)),
        ],
        out_shape=[
            jax.ShapeDtypeStruct((1, 1), jnp.float32),
            jax.ShapeDtypeStruct((1, 1), jnp.float32),
        ],
        scratch_shapes=[pltpu.VMEM((N, D + 2), jnp.float32)],
    )(anchor, positive, negative, embeddings, embeddings)
    return tri_sum[0, 0] / N + ALPHA * (loc_sum[0, 0] / N)


# speedup vs baseline: 5.1490x; 1.0219x over previous
import jax
import jax.numpy as jnp
from jax.experimental import pallas as pl
from jax.experimental.pallas import tpu as pltpu

N = 2048
D = 512
K = 5
BLK = 1024
SUB = 8
NSEL = 16
MARGIN = 1.0
ALPHA = 0.3


def _ce(a, b):
    return jnp.minimum(a, b), jnp.maximum(a, b)


def _bitonic6(L):
    L = list(L)
    for i in range(3):
        L[i], L[i + 3] = _ce(L[i], L[i + 3])
    for base in (0, 3):
        L[base], L[base + 2] = _ce(L[base], L[base + 2])
        L[base], L[base + 1] = _ce(L[base], L[base + 1])
        L[base + 1], L[base + 2] = _ce(L[base + 1], L[base + 2])
    return L


def _merge66(A, B):
    return _bitonic6([jnp.minimum(A[i], B[5 - i]) for i in range(6)])


def _merge44(A, B):
    return _bitonic6([A[0], A[1], jnp.minimum(A[2], B[3]),
                      jnp.minimum(A[3], B[2]), B[1], B[0]])


def _merge22(A, B):
    lo0, hi0 = _ce(A[0], B[0])
    lo1, hi1 = _ce(A[1], B[1])
    mid_lo, mid_hi = _ce(hi0, lo1)
    return [lo0, mid_lo, mid_hi, hi1]


def _top6_sorted_T(sq_t):
    groups = [sq_t[g * SUB:(g + 1) * SUB, :] for g in range(N // SUB)]
    lists = []
    for p in range(len(groups) // 2):
        lo, hi = _ce(groups[2 * p], groups[2 * p + 1])
        lists.append([lo, hi])
    lists = [_merge22(lists[2 * p], lists[2 * p + 1])
             for p in range(len(lists) // 2)]
    lists = [_merge44(lists[2 * p], lists[2 * p + 1])
             for p in range(len(lists) // 2)]
    while len(lists) > 1:
        lists = [_merge66(lists[2 * p], lists[2 * p + 1])
                 for p in range(len(lists) // 2)]
    s = lists[0]
    w = SUB // 2
    while w >= 1:
        s = _merge66([v[:w, :] for v in s], [v[w:2 * w, :] for v in s])
        w //= 2
    return s


def _loss_kernel(a_ref, p_ref, ng_ref, er_ref, e_ref, tri_ref, loc_ref,
                 eaug_ref):
    i = pl.program_id(0)
    ones_d = jnp.ones((D, 1), jnp.float32)

    @pl.when(i == 0)
    def _init():
        tri_ref[...] = jnp.zeros((1, 1), jnp.float32)
        loc_ref[...] = jnp.zeros((1, 1), jnp.float32)
        e0 = e_ref[...]
        cn = jax.lax.dot_general(e0 * e0, ones_d, (((1,), (0,)), ((), ())),
                                 preferred_element_type=jnp.float32)
        eaug_ref[:, :D] = e0 * (-2.0)
        eaug_ref[:, D:D + 1] = cn
        eaug_ref[:, D + 1:D + 2] = jnp.ones((N, 1), jnp.float32)

    a = a_ref[...]
    dp = a - p_ref[...] + 1e-6
    dn = a - ng_ref[...] + 1e-6
    pos_d = jnp.sqrt(jax.lax.dot_general(dp * dp, ones_d,
                                         (((1,), (0,)), ((), ())),
                                         preferred_element_type=jnp.float32))
    neg_d = jnp.sqrt(jax.lax.dot_general(dn * dn, ones_d,
                                         (((1,), (0,)), ((), ())),
                                         preferred_element_type=jnp.float32))
    tri_ref[...] += jnp.sum(jnp.maximum(pos_d - neg_d + MARGIN, 0.0),
                            keepdims=True).reshape(1, 1)

    rows = er_ref[...]
    rn = jax.lax.dot_general(rows * rows, ones_d, (((1,), (0,)), ((), ())),
                             preferred_element_type=jnp.float32)
    rows_aug = jnp.concatenate(
        [rows, jnp.ones((BLK, 1), jnp.float32), rn], axis=1)
    sq_t = jax.lax.dot_general(eaug_ref[...], rows_aug,
                               (((1,), (1,)), ((), ())),
                               preferred_element_type=jnp.float32)

    s6 = _top6_sorted_T(sq_t)
    dm6 = [jnp.sqrt(jnp.maximum(v, 1e-12)) for v in s6]
    nb_sum = dm6[1] + dm6[2] + dm6[3] + dm6[4] + dm6[5]
    t_sq = s6[5]
    count_less = ((s6[0] < t_sq).astype(jnp.float32)
                  + (s6[1] < t_sq).astype(jnp.float32)
                  + (s6[2] < t_sq).astype(jnp.float32)
                  + (s6[3] < t_sq).astype(jnp.float32)
                  + (s6[4] < t_sq).astype(jnp.float32))

    sq16 = sq_t[:NSEL, :]
    dm16 = jnp.sqrt(jnp.maximum(sq16, 1e-12))
    col16 = jax.lax.broadcasted_iota(jnp.int32, (NSEL, BLK), 0)
    row_idx = i * BLK + jax.lax.broadcasted_iota(jnp.int32, (1, BLK), 1)
    is_tie = (sq16 == t_sq).astype(jnp.float32)
    strict_lo = (jax.lax.broadcasted_iota(jnp.int32, (NSEL, NSEL), 1)
                 < jax.lax.broadcasted_iota(jnp.int32, (NSEL, NSEL), 0))
    tie_prefix = jax.lax.dot_general(strict_lo.astype(jnp.float32), is_tie,
                                     (((1,), (0,)), ((), ())),
                                     preferred_element_type=jnp.float32)
    in_top6 = (sq16 < t_sq) | ((sq16 == t_sq)
                               & (tie_prefix + count_less < K + 1))
    valid = ~((col16 == row_idx) | in_top6)
    incl_lo = (jax.lax.broadcasted_iota(jnp.int32, (NSEL, NSEL), 1)
               <= jax.lax.broadcasted_iota(jnp.int32, (NSEL, NSEL), 0))
    rank = jax.lax.dot_general(incl_lo.astype(jnp.float32),
                               valid.astype(jnp.float32),
                               (((1,), (0,)), ((), ())),
                               preferred_element_type=jnp.float32)
    take = valid & (rank <= float(K))
    nn_sum = jax.lax.dot_general(jnp.ones((1, NSEL), jnp.float32),
                                 jnp.where(take, dm16, 0.0),
                                 (((1,), (0,)), ((), ())),
                                 preferred_element_type=jnp.float32)

    loc = jnp.maximum(nb_sum / K - nn_sum / K + 0.1, 0.0)
    loc_ref[...] += jnp.sum(loc, keepdims=True).reshape(1, 1)


def kernel(anchor, positive, negative, embeddings):
    grid = (N // BLK,)
    tri_sum, loc_sum = pl.pallas_call(
        _loss_kernel,
        grid=grid,
        in_specs=[
            pl.BlockSpec((BLK, D), lambda i: (i, 0)),
            pl.BlockSpec((BLK, D), lambda i: (i, 0)),
            pl.BlockSpec((BLK, D), lambda i: (i, 0)),
            pl.BlockSpec((BLK, D), lambda i: (i, 0)),
            pl.BlockSpec((N, D), lambda i: (0, 0)),
        ],
        out_specs=[
            pl.BlockSpec((1, 1), lambda i: (0, 0)),
            pl.BlockSpec((1, 1), lambda i: (0, 0)),
        ],
        out_shape=[
            jax.ShapeDtypeStruct((1, 1), jnp.float32),
            jax.ShapeDtypeStruct((1, 1), jnp.float32),
        ],
        scratch_shapes=[pltpu.VMEM((N, D + 2), jnp.float32)],
    )(anchor, positive, negative, embeddings, embeddings)
    return tri_sum[0, 0] / N + ALPHA * (loc_sum[0, 0] / N)
